# Initial kernel scaffold; baseline (speedup 1.0000x reference)
import jax
import jax.numpy as jnp
from jax.experimental import pallas as pl


def kernel(x, edge_index, W1, b1, W2, b2, Wfc, bfc):
    raise NotImplementedError("write your pallas kernel here")


# speedup vs baseline: 42.8446x; 42.8446x over previous
import functools

import jax
import jax.numpy as jnp
from jax import lax
from jax.experimental import pallas as pl
from jax.experimental.pallas import tpu as pltpu
from jax.experimental.pallas import tpu_sc as plsc

N_NODES = 100000
N_EDGES = 6400000
NP = 100352
EP = 6406144
NCH = EP // 128
CPT1 = NCH // 32
CPT2 = NCH // 16
EPG1 = 4
EPG2 = 8
RPT = NP // 16
BLK = 1024


def _mesh():
    return plsc.VectorSubcoreMesh(core_axis_name="c", subcore_axis_name="s")


def _deg_call(src_chunks, z1):
    @functools.partial(
        pl.kernel,
        out_type=jax.ShapeDtypeStruct((2, NP), jnp.float32),
        mesh=_mesh(),
        compiler_params=pltpu.CompilerParams(use_tc_tiling_on_sc=False),
        scratch_types=[
            pltpu.VMEM((EPG1, 128), jnp.int32),
            pltpu.VMEM((128,), jnp.float32),
            pltpu.VMEM_SHARED((NP,), jnp.float32),
        ],
    )
    def k(srcc, z1_h, out, sidx, ones_v, dacc):
        c = lax.axis_index("c")
        s = lax.axis_index("s")
        r0 = s * RPT
        for i in range(8):
            ones_v[pl.ds(i * 16, 16)] = jnp.ones((16,), jnp.float32)
        pltpu.sync_copy(z1_h.at[pl.ds(r0, RPT)], dacc.at[pl.ds(r0, RPT)])
        plsc.subcore_barrier()
        w = s * 2 + c

        def body(g, carry):
            cid0 = w * CPT1 + g * EPG1
            pltpu.sync_copy(srcc.at[pl.ds(cid0, EPG1)], sidx)
            for j in range(EPG1):
                pltpu.sync_copy(ones_v, dacc.at[sidx.at[j]], add=True)
            return carry

        lax.fori_loop(0, CPT1 // EPG1, body, 0)
        plsc.subcore_barrier()
        pltpu.sync_copy(dacc.at[pl.ds(r0, RPT)], out.at[c, pl.ds(r0, RPT)])

    return k(src_chunks, z1)


def _agg_call(table, src_slabs, dst_chunks, z16, *, split_edges, epg, cpt):
    @functools.partial(
        pl.kernel,
        out_type=jax.ShapeDtypeStruct((2, NP, 16), jnp.float32),
        mesh=_mesh(),
        compiler_params=pltpu.CompilerParams(use_tc_tiling_on_sc=False),
        scratch_types=[
            pltpu.VMEM((epg, 128), jnp.int32),
            pltpu.VMEM((epg, 128), jnp.int32),
            pltpu.VMEM((epg, 128, 16), jnp.float32),
            pltpu.VMEM_SHARED((NP, 16), jnp.float32),
            pltpu.SemaphoreType.DMA,
        ],
    )
    def k(tab, srcc, dstc, z16_h, out, sidx, didx, rows, acc, sem):
        c = lax.axis_index("c")
        s = lax.axis_index("s")
        r0 = s * RPT
        pltpu.sync_copy(z16_h.at[pl.ds(r0, RPT)], acc.at[pl.ds(r0, RPT)])
        plsc.subcore_barrier()
        if split_edges:
            base = (s * 2 + c) * cpt
        else:
            base = s * cpt

        def body(g, carry):
            cid0 = base + g * epg
            if split_edges:
                pltpu.sync_copy(srcc.at[0, pl.ds(cid0, epg)], sidx)
            else:
                pltpu.sync_copy(srcc.at[c, pl.ds(cid0, epg)], sidx)
            pltpu.sync_copy(dstc.at[0, pl.ds(cid0, epg)], didx)
            cps = [
                pltpu.async_copy(tab.at[sidx.at[j]], rows.at[j], sem)
                for j in range(epg)
            ]
            for cp in cps:
                cp.wait()
            for j in range(epg):
                pltpu.sync_copy(rows.at[j], acc.at[didx.at[j]], add=True)
            return carry

        lax.fori_loop(0, cpt // epg, body, 0)
        plsc.subcore_barrier()
        pltpu.sync_copy(acc.at[pl.ds(r0, RPT)], out.at[c, pl.ds(r0, RPT)])

    return k(table, src_slabs, dst_chunks, z16)


def _tc_a(dp, xp):
    def body(dp_ref, x_ref, o_ref):
        deg = dp_ref[0, :] + dp_ref[1, :]
        ns = jnp.where(deg > 0, lax.rsqrt(jnp.maximum(deg, 1.0)), 0.0)
        xs = x_ref[...] * ns[:, None]
        lane = lax.broadcasted_iota(jnp.int32, (BLK, 16), 1)
        o_ref[...] = jnp.where(lane == 10, 1.0, xs)

    return pl.pallas_call(
        body,
        grid=(NP // BLK,),
        in_specs=[
            pl.BlockSpec((2, BLK), lambda i: (0, i)),
            pl.BlockSpec((BLK, 16), lambda i: (i, 0)),
        ],
        out_specs=pl.BlockSpec((BLK, 16), lambda i: (i, 0)),
        out_shape=jax.ShapeDtypeStruct((NP, 16), jnp.float32),
    )(dp, xp)


def _tc_b(a1, dp, W1p, b1p):
    def body(a_ref, dp_ref, w_ref, b_ref, xs2_ref, nd_ref):
        a = a_ref[0] + a_ref[1]
        deg_in = a[:, 10]
        nd = jnp.where(deg_in > 0, lax.rsqrt(jnp.maximum(deg_in, 1.0)), 0.0)
        degp = dp_ref[0, :] + dp_ref[1, :]
        ns = jnp.where(degp > 0, lax.rsqrt(jnp.maximum(degp, 1.0)), 0.0)
        hm = jnp.dot(a, w_ref[...], preferred_element_type=jnp.float32)
        h1 = jnp.maximum(hm * nd[:, None] + b_ref[...], 0.0)
        xs2 = h1 * ns[:, None]
        xs2_ref[0] = xs2[:, :16]
        xs2_ref[1] = xs2[:, 16:]
        nd_ref[...] = nd

    return pl.pallas_call(
        body,
        grid=(NP // BLK,),
        in_specs=[
            pl.BlockSpec((2, BLK, 16), lambda i: (0, i, 0)),
            pl.BlockSpec((2, BLK), lambda i: (0, i)),
            pl.BlockSpec((16, 32), lambda i: (0, 0)),
            pl.BlockSpec((1, 32), lambda i: (0, 0)),
        ],
        out_specs=[
            pl.BlockSpec((2, BLK, 16), lambda i: (0, i, 0)),
            pl.BlockSpec((BLK,), lambda i: (i,)),
        ],
        out_shape=[
            jax.ShapeDtypeStruct((2, NP, 16), jnp.float32),
            jax.ShapeDtypeStruct((NP,), jnp.float32),
        ],
    )(a1, dp, W1p, b1p)


def _tc_c(a2, nd, W2a, W2b, b2p, Wfcp, bfcp):
    def body(a_ref, nd_ref, w2a_ref, w2b_ref, b2_ref, wfc_ref, bfc_ref, o_ref):
        hm = jnp.dot(a_ref[0], w2a_ref[...], preferred_element_type=jnp.float32)
        hm += jnp.dot(a_ref[1], w2b_ref[...], preferred_element_type=jnp.float32)
        nd = nd_ref[...]
        h2 = jnp.maximum(hm * nd[:, None] + b2_ref[...], 0.0)
        o = jnp.dot(h2, wfc_ref[...], preferred_element_type=jnp.float32)
        o_ref[...] = o + bfc_ref[...]

    return pl.pallas_call(
        body,
        grid=(NP // BLK,),
        in_specs=[
            pl.BlockSpec((2, BLK, 16), lambda i: (0, i, 0)),
            pl.BlockSpec((BLK,), lambda i: (i,)),
            pl.BlockSpec((16, 32), lambda i: (0, 0)),
            pl.BlockSpec((16, 32), lambda i: (0, 0)),
            pl.BlockSpec((1, 32), lambda i: (0, 0)),
            pl.BlockSpec((32, 16), lambda i: (0, 0)),
            pl.BlockSpec((1, 16), lambda i: (0, 0)),
        ],
        out_specs=pl.BlockSpec((BLK, 16), lambda i: (i, 0)),
        out_shape=jax.ShapeDtypeStruct((NP, 16), jnp.float32),
    )(a2, nd, W2a, W2b, b2p, Wfcp, bfcp)


def kernel(x, edge_index, W1, b1, W2, b2, Wfc, bfc):
    ei = edge_index.astype(jnp.int32)
    pad = jnp.full((EP - N_EDGES,), N_NODES, jnp.int32)
    src = jnp.concatenate([ei[0], pad]).reshape(NCH, 128)
    dst = jnp.concatenate([ei[1], pad]).reshape(1, NCH, 128)

    xp = jnp.pad(x, ((0, NP - N_NODES), (0, 16 - x.shape[1])))
    z1 = jnp.zeros((NP,), jnp.float32)
    z16 = jnp.zeros((NP, 16), jnp.float32)
    W1p = jnp.pad(W1, ((0, 6), (0, 12)))
    b1p = jnp.pad(b1, (0, 12)).reshape(1, 32)
    W2p = jnp.pad(W2, ((0, 12), (0, 2)))
    W2a, W2b = W2p[:16], W2p[16:]
    b2p = jnp.pad(b2, (0, 2)).reshape(1, 32)
    Wfcp = jnp.pad(Wfc, ((0, 2), (0, 6)))
    bfcp = jnp.pad(bfc, (0, 6)).reshape(1, 16)

    dp = _deg_call(src, z1)
    xs1 = _tc_a(dp, xp)
    a1 = _agg_call(xs1, src[None], dst, z16,
                   split_edges=True, epg=EPG1, cpt=CPT1)
    xs2, nd = _tc_b(a1, dp, W1p, b1p)
    xs2cat = xs2.reshape(2 * NP, 16)
    src2 = jnp.stack([src, src + NP])
    a2 = _agg_call(xs2cat, src2, dst, z16,
                   split_edges=False, epg=EPG2, cpt=CPT2)
    out = _tc_c(a2, nd, W2a, W2b, b2p, Wfcp, bfcp)
    return out[:N_NODES, :10]


# speedup vs baseline: 57.3593x; 1.3388x over previous
import functools

import jax
import jax.numpy as jnp
from jax import lax
from jax.experimental import pallas as pl
from jax.experimental.pallas import tpu as pltpu
from jax.experimental.pallas import tpu_sc as plsc

N_NODES = 100000
N_EDGES = 6400000
NP = 100352
EP = 6488064
NCH = EP // 128
CPT1 = NCH // 32
CPT2 = NCH // 16
EPG1 = 8
RPT = NP // 16
BLK = 1024


def _mesh():
    return plsc.VectorSubcoreMesh(core_axis_name="c", subcore_axis_name="s")


def _deg_call(eint, z1):
    @functools.partial(
        pl.kernel,
        out_type=jax.ShapeDtypeStruct((2, NP), jnp.float32),
        mesh=_mesh(),
        compiler_params=pltpu.CompilerParams(use_tc_tiling_on_sc=False),
        scratch_types=[
            pltpu.VMEM((2, EPG1, 2, 128), jnp.int32),
            pltpu.VMEM((128,), jnp.float32),
            pltpu.VMEM_SHARED((NP,), jnp.float32),
            pltpu.SemaphoreType.DMA,
            pltpu.SemaphoreType.DMA,
        ],
    )
    def k(eint_h, z1_h, out, sidx, ones_v, dacc, sem0, sem1):
        c = lax.axis_index("c")
        s = lax.axis_index("s")
        r0 = s * RPT
        sems = (sem0, sem1)
        ng = CPT1 // EPG1
        for i in range(8):
            ones_v[pl.ds(i * 16, 16)] = jnp.ones((16,), jnp.float32)
        pltpu.sync_copy(z1_h.at[pl.ds(r0, RPT)], dacc.at[pl.ds(r0, RPT)])
        plsc.subcore_barrier()
        base = (s * 2 + c) * CPT1

        def phase(g, bank):
            @pl.when(g >= 2)
            def _():
                for j in range(EPG1):
                    pltpu.make_async_copy(
                        ones_v, dacc.at[sidx.at[bank, j, 0]], sems[bank]
                    ).wait()

            pltpu.sync_copy(eint_h.at[pl.ds(base + g * EPG1, EPG1)],
                            sidx.at[bank])
            for j in range(EPG1):
                pltpu.async_copy(ones_v, dacc.at[sidx.at[bank, j, 0]],
                                 sems[bank], add=True)

        def body(rr, carry):
            phase(2 * rr, 0)
            phase(2 * rr + 1, 1)
            return carry

        lax.fori_loop(0, ng // 2, body, 0)
        for bank in (0, 1):
            for j in range(EPG1):
                pltpu.make_async_copy(
                    ones_v, dacc.at[sidx.at[bank, j, 0]], sems[bank]
                ).wait()
        plsc.subcore_barrier()
        pltpu.sync_copy(dacc.at[pl.ds(r0, RPT)], out.at[c, pl.ds(r0, RPT)])

    return k(eint, z1)


def _agg_call(tab, eint, zw, *, split_edges, cpt, w, epg):
    @functools.partial(
        pl.kernel,
        out_type=jax.ShapeDtypeStruct((2, NP, w), jnp.float32),
        mesh=_mesh(),
        compiler_params=pltpu.CompilerParams(use_tc_tiling_on_sc=False),
        scratch_types=[
            pltpu.VMEM((3, epg, 2, 128), jnp.int32),
            pltpu.VMEM((3, epg, 128, w), jnp.float32),
            pltpu.VMEM_SHARED((NP, w), jnp.float32),
        ] + [pltpu.SemaphoreType.DMA] * 6,
    )
    def k(tab_h, eint_h, z16_h, out, sidx, rows, acc,
          gsem0, gsem1, gsem2, ssem0, ssem1, ssem2):
        c = lax.axis_index("c")
        s = lax.axis_index("s")
        r0 = s * RPT
        gsems = (gsem0, gsem1, gsem2)
        ssems = (ssem0, ssem1, ssem2)
        ng = cpt // epg
        pltpu.sync_copy(z16_h.at[pl.ds(r0, RPT)], acc.at[pl.ds(r0, RPT)])
        plsc.subcore_barrier()
        if split_edges:
            base = (s * 2 + c) * cpt
        else:
            base = s * cpt
        off = c * NP

        def stage_and_fire(g, bank):
            pltpu.sync_copy(eint_h.at[pl.ds(base + g * epg, epg)],
                            sidx.at[bank])
            if not split_edges:
                for j in range(epg):
                    for v in range(8):
                        sl = (bank, j, 0, pl.ds(v * 16, 16))
                        sidx[sl] = sidx[sl] + off
            for j in range(epg):
                pltpu.async_copy(tab_h.at[sidx.at[bank, j, 0]],
                                 rows.at[bank, j], gsems[bank])

        def drain_scatters(bank):
            for j in range(epg):
                pltpu.make_async_copy(rows.at[bank, j],
                                      acc.at[sidx.at[bank, j, 1]],
                                      ssems[bank]).wait()

        def phase(g, bank):
            for j in range(epg):
                pltpu.make_async_copy(tab_h.at[sidx.at[bank, j, 0]],
                                      rows.at[bank, j], gsems[bank]).wait()
            for j in range(epg):
                pltpu.async_copy(rows.at[bank, j], acc.at[sidx.at[bank, j, 1]],
                                 ssems[bank], add=True)
            b2 = (bank + 2) % 3

            @pl.when(g >= 1)
            def _():
                drain_scatters(b2)

            @pl.when(g + 2 < ng)
            def _():
                stage_and_fire(g + 2, b2)

        stage_and_fire(0, 0)
        stage_and_fire(1, 1)

        def body(rr, carry):
            phase(3 * rr, 0)
            phase(3 * rr + 1, 1)
            phase(3 * rr + 2, 2)
            return carry

        lax.fori_loop(0, ng // 3, body, 0)
        drain_scatters((ng - 1) % 3)
        plsc.subcore_barrier()
        pltpu.sync_copy(acc.at[pl.ds(r0, RPT)], out.at[c, pl.ds(r0, RPT)])

    return k(tab, eint, zw)


def _tc_a(dp, xp):
    def body(dp_ref, x_ref, o_ref):
        deg = dp_ref[0, :] + dp_ref[1, :]
        ns = jnp.where(deg > 0, lax.rsqrt(jnp.maximum(deg, 1.0)), 0.0)
        xs = x_ref[...] * ns[:, None]
        lane = lax.broadcasted_iota(jnp.int32, (BLK, 16), 1)
        o_ref[...] = jnp.where(lane == 10, 1.0, xs)

    return pl.pallas_call(
        body,
        grid=(NP // BLK,),
        in_specs=[
            pl.BlockSpec((2, BLK), lambda i: (0, i)),
            pl.BlockSpec((BLK, 16), lambda i: (i, 0)),
        ],
        out_specs=pl.BlockSpec((BLK, 16), lambda i: (i, 0)),
        out_shape=jax.ShapeDtypeStruct((NP, 16), jnp.float32),
    )(dp, xp)


def _tc_b(a1, dp, W1p, b1p):
    def body(a_ref, dp_ref, w_ref, b_ref, xs2_ref, nd_ref):
        a = a_ref[0] + a_ref[1]
        deg_in = a[:, 10]
        nd = jnp.where(deg_in > 0, lax.rsqrt(jnp.maximum(deg_in, 1.0)), 0.0)
        degp = dp_ref[0, :] + dp_ref[1, :]
        ns = jnp.where(degp > 0, lax.rsqrt(jnp.maximum(degp, 1.0)), 0.0)
        hm = jnp.dot(a, w_ref[...], preferred_element_type=jnp.float32)
        h1 = jnp.maximum(hm * nd[:, None] + b_ref[...], 0.0)
        xs2 = h1 * ns[:, None]
        xs2_ref[0] = xs2[:, :16]
        xs2_ref[1] = xs2[:, 16:]
        nd_ref[...] = nd

    return pl.pallas_call(
        body,
        grid=(NP // BLK,),
        in_specs=[
            pl.BlockSpec((2, BLK, 16), lambda i: (0, i, 0)),
            pl.BlockSpec((2, BLK), lambda i: (0, i)),
            pl.BlockSpec((16, 32), lambda i: (0, 0)),
            pl.BlockSpec((1, 32), lambda i: (0, 0)),
        ],
        out_specs=[
            pl.BlockSpec((2, BLK, 16), lambda i: (0, i, 0)),
            pl.BlockSpec((BLK,), lambda i: (i,)),
        ],
        out_shape=[
            jax.ShapeDtypeStruct((2, NP, 16), jnp.float32),
            jax.ShapeDtypeStruct((NP,), jnp.float32),
        ],
    )(a1, dp, W1p, b1p)


def _tc_c(a2, nd, W2a, W2b, b2p, Wfcp, bfcp):
    def body(a_ref, nd_ref, w2a_ref, w2b_ref, b2_ref, wfc_ref, bfc_ref, o_ref):
        hm = jnp.dot(a_ref[0], w2a_ref[...], preferred_element_type=jnp.float32)
        hm += jnp.dot(a_ref[1], w2b_ref[...], preferred_element_type=jnp.float32)
        nd = nd_ref[...]
        h2 = jnp.maximum(hm * nd[:, None] + b2_ref[...], 0.0)
        o = jnp.dot(h2, wfc_ref[...], preferred_element_type=jnp.float32)
        o_ref[...] = o + bfc_ref[...]

    return pl.pallas_call(
        body,
        grid=(NP // BLK,),
        in_specs=[
            pl.BlockSpec((2, BLK, 16), lambda i: (0, i, 0)),
            pl.BlockSpec((BLK,), lambda i: (i,)),
            pl.BlockSpec((16, 32), lambda i: (0, 0)),
            pl.BlockSpec((16, 32), lambda i: (0, 0)),
            pl.BlockSpec((1, 32), lambda i: (0, 0)),
            pl.BlockSpec((32, 16), lambda i: (0, 0)),
            pl.BlockSpec((1, 16), lambda i: (0, 0)),
        ],
        out_specs=pl.BlockSpec((BLK, 16), lambda i: (i, 0)),
        out_shape=jax.ShapeDtypeStruct((NP, 16), jnp.float32),
    )(a2, nd, W2a, W2b, b2p, Wfcp, bfcp)


def kernel(x, edge_index, W1, b1, W2, b2, Wfc, bfc):
    ei = edge_index.astype(jnp.int32)
    eip = jnp.pad(ei, ((0, 0), (0, EP - N_EDGES)), constant_values=N_NODES)
    eint = eip.reshape(2, NCH, 128).transpose(1, 0, 2)

    xp = jnp.pad(x, ((0, NP - N_NODES), (0, 6)))
    z1 = jnp.zeros((NP,), jnp.float32)
    z16 = jnp.zeros((NP, 16), jnp.float32)
    W1p = jnp.pad(W1, ((0, 6), (0, 12)))
    b1p = jnp.pad(b1, (0, 12)).reshape(1, 32)
    W2p = jnp.pad(W2, ((0, 12), (0, 2)))
    W2a, W2b = W2p[:16], W2p[16:]
    b2p = jnp.pad(b2, (0, 2)).reshape(1, 32)
    Wfcp = jnp.pad(Wfc, ((0, 2), (0, 6)))
    bfcp = jnp.pad(bfc, (0, 6)).reshape(1, 16)

    dp = _deg_call(eint, z1)
    xs1 = _tc_a(dp, xp)
    a1 = _agg_call(xs1, eint, z16, split_edges=True, cpt=CPT1, w=16, epg=4)
    xs2, nd = _tc_b(a1, dp, W1p, b1p)
    a2 = _agg_call(xs2.reshape(2 * NP, 16), eint, z16,
                   split_edges=False, cpt=CPT2, w=16, epg=4)
    out = _tc_c(a2, nd, W2a, W2b, b2p, Wfcp, bfcp)
    return out[:N_NODES, :10]
